# baseline (speedup 1.0000x reference)
import functools

import jax
import jax.numpy as jnp
from jax import lax
from jax.experimental import pallas as pl
from jax.experimental.pallas import tpu as pltpu
from jax.experimental.pallas import tpu_sc as plsc

_SHIFT = 44100
_B, _S, _C, _T = 8, 4, 2, 441000
_L = _T - _SHIFT
_NW = 32
_LT = (_L // 128) * 128
_LREM = _L - _LT
_M = 21376
_NFULL = _LT // _M
_TAILC = _LT - _NFULL * _M
_NCH = _NFULL + 1


def _rotate(a, b, s, idxvec, selmask):
    src = jnp.where(selmask, b, a)
    return jnp.take_along_axis(src, idxvec, axis=0, mode="promise_in_bounds")


def _sc_impl(wav3, offs):
    mesh = plsc.VectorSubcoreMesh(core_axis_name="c", subcore_axis_name="s")

    @functools.partial(
        pl.kernel,
        mesh=mesh,
        out_type=[
            jax.ShapeDtypeStruct((_NW, _C, _L), jnp.float32),
            jax.ShapeDtypeStruct((_NW, _C, _LT + 128), jnp.float32),
        ],
        scratch_types=[
            pltpu.VMEM((48,), jnp.int32),
            pltpu.VMEM((_C, _M + 128), jnp.float32),
            pltpu.VMEM((_C, _M + 128), jnp.float32),
            pltpu.VMEM((_C, _M + 128), jnp.float32),
            pltpu.VMEM((_C, 256), jnp.float32),
            pltpu.SemaphoreType.DMA,
            pltpu.SemaphoreType.DMA,
            pltpu.SemaphoreType.DMA,
            pltpu.SemaphoreType.DMA,
            pltpu.SemaphoreType.DMA,
            pltpu.SemaphoreType.DMA,
            pltpu.SemaphoreType.DMA,
            pltpu.SemaphoreType.DMA,
        ],
    )
    def k(wav_hbm, off_hbm, out_hbm, tails_hbm, off_v, b0, b1, b2, traw,
          g0, g1, g2, s0, s1, s2, tg, ts):
        cid = lax.axis_index("c")
        sid = lax.axis_index("s")
        w = cid * 16 + sid

        pltpu.sync_copy(off_hbm.at[pl.ds(0, 32)], off_v.at[pl.ds(0, 32)])
        off = off_v[pl.ds(w, 16)][0]

        col0 = (off // 128) * 128
        phi = off - col0
        s = phi % 16
        phi16 = pl.multiple_of(phi - s, 16)
        lanes = lax.iota(jnp.int32, 16)
        idxvec = (lanes + s) & 15
        selmask = lanes < s

        slots = (b0, b1, b2)
        gsem = (g0, g1, g2)
        ssem = (s0, s1, s2)

        def chunk_cols(j):
            return _M if j < _NFULL else _TAILC

        def issue_gather(j):
            mlen = chunk_cols(j)
            return pltpu.async_copy(
                wav_hbm.at[w, :, pl.ds(pl.multiple_of(col0 + j * _M, 128),
                                       mlen + 128)],
                slots[j % 3].at[:, pl.ds(0, mlen + 128)], gsem[j % 3])

        def realign(buf, mlen):
            def body(i, carry):
                base = i * 64
                nxt = []
                for c in range(_C):
                    cur = carry[c]
                    for u in range(4):
                        b = buf[c, pl.ds(phi16 + base + u * 16 + 16, 16)]
                        buf[c, pl.ds(base + u * 16, 16)] = _rotate(
                            cur, b, s, idxvec, selmask)
                        cur = b
                    nxt.append(cur)
                return tuple(nxt)

            init = tuple(buf[c, pl.ds(phi16, 16)] for c in range(_C))
            lax.fori_loop(0, mlen // 64, body, init)

        def issue_scatter(j):
            mlen = chunk_cols(j)
            return pltpu.async_copy(
                slots[j % 3].at[:, pl.ds(0, mlen)],
                out_hbm.at[w, :, pl.ds(j * _M, mlen)], ssem[j % 3])

        tail_h = pltpu.async_copy(
            wav_hbm.at[w, :, pl.ds(pl.multiple_of(col0 + _LT, 128), 256)],
            traw, tg)

        gh = [None] * _NCH
        sh = [None] * _NCH
        gh[0] = issue_gather(0)
        gh[1] = issue_gather(1)
        for j in range(_NCH):
            gh[j].wait()
            realign(slots[j % 3], chunk_cols(j))
            sh[j] = issue_scatter(j)
            if j + 2 < _NCH:
                if j - 1 >= 0:
                    sh[j - 1].wait()
                    sh[j - 1] = None
                gh[j + 2] = issue_gather(j + 2)

        tail_h.wait()
        realign(traw, 128)
        pltpu.async_copy(traw.at[:, pl.ds(0, 128)],
                         tails_hbm.at[w, :, pl.ds(_LT, 128)], ts).wait()
        for h in sh:
            if h is not None:
                h.wait()

    return k(wav3, offs)


def _tc_patch(main, tails):
    def patch(main_any, tails_ref, out_ref):
        del main_any
        out_ref[...] = tails_ref[...]

    return pl.pallas_call(
        patch,
        grid=(1,),
        in_specs=[
            pl.BlockSpec(memory_space=pl.ANY),
            pl.BlockSpec((_NW, _C, 128), lambda i: (0, 0, _LT // 128)),
        ],
        out_specs=pl.BlockSpec((_NW, _C, 128), lambda i: (0, 0, _LT // 128)),
        out_shape=jax.ShapeDtypeStruct((_NW, _C, _L), jnp.float32),
        input_output_aliases={0: 0},
    )(main, tails)


def kernel(wav, offsets):
    wav3 = wav.reshape(_NW, _C, _T)
    offs = offsets.reshape(_NW).astype(jnp.int32)
    main, tails = _sc_impl(wav3, offs)
    out = _tc_patch(main, tails)
    return out.reshape(_B, _S, _C, _L)
